# baseline (speedup 1.0000x reference)
import functools

import jax
import jax.numpy as jnp
from jax import lax
from jax.experimental import pallas as pl
from jax.experimental.pallas import tpu as pltpu
from jax.experimental.pallas import tpu_sc as plsc


_T = 8192
_D = 1024
_INFO = plsc.get_sparse_core_info()
_NW = _INFO.num_cores * _INFO.num_subcores
_ROWS_PER_W = _T // _NW
_CHUNK = 16
_NBUF = 4
_NCHUNKS = _ROWS_PER_W // _CHUNK


@functools.partial(
    pl.kernel,
    mesh=plsc.VectorSubcoreMesh(core_axis_name="c", subcore_axis_name="s"),
    out_type=jax.ShapeDtypeStruct((1, _T, _D), jnp.float32),
    scratch_types=[
        pltpu.VMEM((_NBUF, _CHUNK, _D), jnp.float32),
        pltpu.SemaphoreType.DMA((_NBUF,)),
        pltpu.SemaphoreType.DMA((_NBUF,)),
    ],
)
def _sc_copy(emb_hbm, out_hbm, buf, in_sems, out_sems):
    wid = lax.axis_index("s") * _INFO.num_cores + lax.axis_index("c")
    base = wid * _ROWS_PER_W

    def in_copy(i, slot):
        return pltpu.make_async_copy(
            emb_hbm.at[pl.ds(base + i * _CHUNK, _CHUNK), :],
            buf.at[slot],
            in_sems.at[slot],
        )

    def out_copy(i, slot):
        return pltpu.make_async_copy(
            buf.at[slot],
            out_hbm.at[0, pl.ds(base + i * _CHUNK, _CHUNK), :],
            out_sems.at[slot],
        )

    lag = _NBUF // 2
    for i in range(min(_NBUF, _NCHUNKS)):
        in_copy(i, i).start()
    for i in range(_NCHUNKS + lag):
        if i < _NCHUNKS:
            slot = i % _NBUF
            in_copy(i, slot).wait()
            out_copy(i, slot).start()
        j = i - lag
        if 0 <= j and j + _NBUF < _NCHUNKS:
            out_copy(j, j % _NBUF).wait()
            in_copy(j + _NBUF, j % _NBUF).start()
    for i in range(max(0, _NCHUNKS - _NBUF), _NCHUNKS):
        out_copy(i, i % _NBUF).wait()


def kernel(x, emb):
    del x
    return _sc_copy(emb)


# speedup vs baseline: 1.0289x; 1.0289x over previous
import functools

import jax
import jax.numpy as jnp
from jax import lax
from jax.experimental import pallas as pl
from jax.experimental.pallas import tpu as pltpu
from jax.experimental.pallas import tpu_sc as plsc


_CHUNK = 32
_NBUF = 2


@functools.lru_cache(maxsize=None)
def _make_sc_copy(T, D):
    info = plsc.get_sparse_core_info()
    nw = info.num_cores * info.num_subcores
    rows_per_w = T // nw
    chunk = min(_CHUNK, rows_per_w)
    nchunks = rows_per_w // chunk
    assert T % nw == 0 and rows_per_w % chunk == 0

    @functools.partial(
        pl.kernel,
        mesh=plsc.VectorSubcoreMesh(core_axis_name="c", subcore_axis_name="s"),
        out_type=jax.ShapeDtypeStruct((1, T, D), jnp.float32),
        scratch_types=[
            pltpu.VMEM((_NBUF, chunk, D), jnp.float32),
            pltpu.SemaphoreType.DMA((_NBUF,)),
            pltpu.SemaphoreType.DMA((_NBUF,)),
        ],
    )
    def sc_copy(emb_hbm, out_hbm, buf, in_sems, out_sems):
        wid = lax.axis_index("s") * info.num_cores + lax.axis_index("c")
        base = wid * rows_per_w

        def in_copy(i, slot):
            return pltpu.make_async_copy(
                emb_hbm.at[pl.ds(base + i * chunk, chunk), :],
                buf.at[slot],
                in_sems.at[slot],
            )

        def out_copy(i, slot):
            return pltpu.make_async_copy(
                buf.at[slot],
                out_hbm.at[0, pl.ds(base + i * chunk, chunk), :],
                out_sems.at[slot],
            )

        in_copy(0, 0).start()
        for i in range(nchunks):
            slot = i % _NBUF
            if i + 1 < nchunks:
                nslot = (i + 1) % _NBUF
                if i + 1 >= _NBUF:
                    out_copy(i + 1 - _NBUF, nslot).wait()
                in_copy(i + 1, nslot).start()
            in_copy(i, slot).wait()
            out_copy(i, slot).start()
        for i in range(max(0, nchunks - _NBUF), nchunks):
            out_copy(i, i % _NBUF).wait()

    return sc_copy


def kernel(x, emb):
    del x
    T, D = emb.shape
    return _make_sc_copy(T, D)(emb)
